# baseline (speedup 1.0000x reference)
import functools

import jax
import jax.numpy as jnp
from jax import lax
from jax.experimental import pallas as pl
from jax.experimental.pallas import tpu as pltpu
from jax.experimental.pallas import tpu_sc as plsc

BATCH = 16384
N_FIELDS = 26
FIELD_DIM = 100000
TOTAL_ROWS = N_FIELDS * FIELD_DIM

SPLIT = (TOTAL_ROWS // 1024) * 1024
TAIL = TOTAL_ROWS - SPLIT

NUM_CORES = 2
NUM_SUBCORES = 16
NUM_WORKERS = NUM_CORES * NUM_SUBCORES
ROWS_PER_W = BATCH // NUM_WORKERS
FLAT_PER_W = ROWS_PER_W * N_FIELDS
LANES = 16

_mesh = plsc.VectorSubcoreMesh(core_axis_name="c", subcore_axis_name="s")

_N_COPY_CHUNKS = 16
_chunk_tiles = SPLIT // 1024 // _N_COPY_CHUNKS
_COPY_SIZES = [_chunk_tiles * 1024] * (_N_COPY_CHUNKS - 1)
_COPY_SIZES.append(SPLIT - sum(_COPY_SIZES))
_COPY_STARTS = [sum(_COPY_SIZES[:k]) for k in range(_N_COPY_CHUNKS)]


def _linearize_body(src_ref, dst_ref, *sems):
    copies = [
        pltpu.make_async_copy(
            src_ref.at[0, pl.ds(_COPY_STARTS[k], _COPY_SIZES[k])],
            dst_ref.at[pl.ds(_COPY_STARTS[k], _COPY_SIZES[k])],
            sems[k],
        )
        for k in range(_N_COPY_CHUNKS)
    ]
    for c in copies:
        c.start()
    for c in copies:
        c.wait()


_w_linearize = pl.pallas_call(
    _linearize_body,
    out_shape=jax.ShapeDtypeStruct((SPLIT,), jnp.float32),
    in_specs=[pl.BlockSpec(memory_space=pl.ANY)],
    out_specs=pl.BlockSpec(memory_space=pl.ANY),
    scratch_shapes=[pltpu.SemaphoreType.DMA] * _N_COPY_CHUNKS,
)


@functools.partial(
    pl.kernel,
    mesh=_mesh,
    out_type=jax.ShapeDtypeStruct((BATCH,), jnp.float32),
    compiler_params=pltpu.CompilerParams(needs_layout_passes=False),
    scratch_types=[
        pltpu.VMEM((FLAT_PER_W,), jnp.int32),
        pltpu.VMEM((FLAT_PER_W,), jnp.float32),
        pltpu.VMEM((ROWS_PER_W,), jnp.int32),
        pltpu.VMEM((TAIL,), jnp.float32),
        pltpu.VMEM((ROWS_PER_W,), jnp.float32),
        pltpu.VMEM((LANES,), jnp.float32),
        pltpu.SemaphoreType.DMA,
    ],
)
def _lr_sc(idx_hbm, tdelta_hbm, w_hbm, wtail_hbm, bias_hbm, out_hbm,
           idx_v, vals_v, tdelta_v, wtail_v, out_v, bias_v, sem):
    wid = lax.axis_index("s") * NUM_CORES + lax.axis_index("c")
    base = wid * FLAT_PER_W
    rbase = wid * ROWS_PER_W

    pltpu.sync_copy(idx_hbm.at[pl.ds(base, FLAT_PER_W)], idx_v)
    pltpu.sync_copy(tdelta_hbm.at[pl.ds(rbase, ROWS_PER_W)], tdelta_v)
    pltpu.sync_copy(wtail_hbm, wtail_v)
    pltpu.sync_copy(bias_hbm, bias_v)

    pltpu.async_copy(w_hbm.at[idx_v], vals_v, sem).wait()

    lane = lax.iota(jnp.int32, LANES)
    bvec = bias_v[...]
    row16 = lane * N_FIELDS
    zero16 = jnp.zeros((LANES,), jnp.int32)

    def reduce_block(blk, carry):
        b0 = blk * (LANES * N_FIELDS)
        acc = bvec
        for f in range(N_FIELDS - 1):
            acc = acc + plsc.load_gather(vals_v, [row16 + (b0 + f)])
        g = plsc.load_gather(vals_v, [row16 + (b0 + (N_FIELDS - 1))])
        d = tdelta_v[pl.ds(blk * LANES, LANES)]
        tv = plsc.load_gather(wtail_v, [jnp.maximum(d - 1, zero16)])
        acc = acc + jnp.where(d > 0, tv, g)
        out_v[pl.ds(blk * LANES, LANES)] = 1.0 / (1.0 + jnp.exp(-acc))
        return carry

    lax.fori_loop(0, ROWS_PER_W // LANES, reduce_block, 0)

    pltpu.sync_copy(out_v, out_hbm.at[pl.ds(rbase, ROWS_PER_W)])


def kernel(data, W, bias):
    offsets = jnp.arange(N_FIELDS, dtype=data.dtype) * FIELD_DIM
    idxg = data + offsets[None, :]
    idx_safe = jnp.minimum(idxg, SPLIT - 1).reshape(-1).astype(jnp.int32)
    last = data[:, N_FIELDS - 1]
    tdelta = jnp.where(
        last >= FIELD_DIM - TAIL, last - (FIELD_DIM - TAIL) + 1, 0
    ).astype(jnp.int32)
    w_main = _w_linearize(W.T)
    w_tail = lax.slice(W, (SPLIT, 0), (TOTAL_ROWS, 1)).reshape(-1)
    bias16 = jnp.broadcast_to(bias.astype(jnp.float32), (LANES,))
    return _lr_sc(idx_safe, tdelta, w_main, w_tail, bias16)


# speedup vs baseline: 5.0893x; 5.0893x over previous
import functools

import jax
import jax.numpy as jnp
from jax import lax
from jax.experimental import pallas as pl
from jax.experimental.pallas import tpu as pltpu
from jax.experimental.pallas import tpu_sc as plsc

BATCH = 16384
N_FIELDS = 26
FIELD_DIM = 100000
TOTAL_ROWS = N_FIELDS * FIELD_DIM

SPLIT = (TOTAL_ROWS // 1024) * 1024
TAIL = TOTAL_ROWS - SPLIT

NUM_CORES = 2
NUM_SUBCORES = 16
NUM_WORKERS = NUM_CORES * NUM_SUBCORES
ROWS_PER_W = BATCH // NUM_WORKERS
FLAT_PER_W = ROWS_PER_W * N_FIELDS
LANES = 16

_mesh = plsc.VectorSubcoreMesh(core_axis_name="c", subcore_axis_name="s")

_RETILE_BLOCK = 131072
_RETILE_GRID = -(-SPLIT // _RETILE_BLOCK)


def _retile_body(src_ref, dst_ref):
    dst_ref[...] = src_ref[0]


_w_linearize = pl.pallas_call(
    _retile_body,
    grid=(_RETILE_GRID,),
    out_shape=jax.ShapeDtypeStruct((SPLIT,), jnp.float32),
    in_specs=[pl.BlockSpec((1, _RETILE_BLOCK), lambda i: (0, i))],
    out_specs=pl.BlockSpec((_RETILE_BLOCK,), lambda i: (i,)),
)


@functools.partial(
    pl.kernel,
    mesh=_mesh,
    out_type=jax.ShapeDtypeStruct((BATCH,), jnp.float32),
    compiler_params=pltpu.CompilerParams(needs_layout_passes=False),
    scratch_types=[
        pltpu.VMEM((FLAT_PER_W,), jnp.int32),
        pltpu.VMEM((FLAT_PER_W,), jnp.float32),
        pltpu.VMEM((ROWS_PER_W,), jnp.int32),
        pltpu.VMEM((TAIL,), jnp.float32),
        pltpu.VMEM((ROWS_PER_W,), jnp.float32),
        pltpu.VMEM((LANES,), jnp.float32),
        pltpu.SemaphoreType.DMA,
    ],
)
def _lr_sc(idx_hbm, tdelta_hbm, w_hbm, wtail_hbm, bias_hbm, out_hbm,
           idx_v, vals_v, tdelta_v, wtail_v, out_v, bias_v, sem):
    wid = lax.axis_index("s") * NUM_CORES + lax.axis_index("c")
    base = wid * FLAT_PER_W
    rbase = wid * ROWS_PER_W

    pltpu.sync_copy(idx_hbm.at[pl.ds(base, FLAT_PER_W)], idx_v)
    pltpu.sync_copy(tdelta_hbm.at[pl.ds(rbase, ROWS_PER_W)], tdelta_v)
    pltpu.sync_copy(wtail_hbm, wtail_v)
    pltpu.sync_copy(bias_hbm, bias_v)

    pltpu.async_copy(w_hbm.at[idx_v], vals_v, sem).wait()

    lane = lax.iota(jnp.int32, LANES)
    bvec = bias_v[...]
    row16 = lane * N_FIELDS
    zero16 = jnp.zeros((LANES,), jnp.int32)

    def reduce_block(blk, carry):
        b0 = blk * (LANES * N_FIELDS)
        acc = bvec
        for f in range(N_FIELDS - 1):
            acc = acc + plsc.load_gather(vals_v, [row16 + (b0 + f)])
        g = plsc.load_gather(vals_v, [row16 + (b0 + (N_FIELDS - 1))])
        d = tdelta_v[pl.ds(blk * LANES, LANES)]
        tv = plsc.load_gather(wtail_v, [jnp.maximum(d - 1, zero16)])
        acc = acc + jnp.where(d > 0, tv, g)
        out_v[pl.ds(blk * LANES, LANES)] = 1.0 / (1.0 + jnp.exp(-acc))
        return carry

    lax.fori_loop(0, ROWS_PER_W // LANES, reduce_block, 0)

    pltpu.sync_copy(out_v, out_hbm.at[pl.ds(rbase, ROWS_PER_W)])


def kernel(data, W, bias):
    offsets = jnp.arange(N_FIELDS, dtype=data.dtype) * FIELD_DIM
    idxg = data + offsets[None, :]
    idx_safe = jnp.minimum(idxg, SPLIT - 1).reshape(-1).astype(jnp.int32)
    last = data[:, N_FIELDS - 1]
    tdelta = jnp.where(
        last >= FIELD_DIM - TAIL, last - (FIELD_DIM - TAIL) + 1, 0
    ).astype(jnp.int32)
    w_main = _w_linearize(W.T)
    w_tail = lax.slice(W, (SPLIT, 0), (TOTAL_ROWS, 1)).reshape(-1)
    bias16 = jnp.broadcast_to(bias.astype(jnp.float32), (LANES,))
    return _lr_sc(idx_safe, tdelta, w_main, w_tail, bias16)


# speedup vs baseline: 6.4180x; 1.2611x over previous
import functools

import jax
import jax.numpy as jnp
from jax import lax
from jax.experimental import pallas as pl
from jax.experimental.pallas import tpu as pltpu
from jax.experimental.pallas import tpu_sc as plsc

BATCH = 16384
N_FIELDS = 26
FIELD_DIM = 100000
TOTAL_ROWS = N_FIELDS * FIELD_DIM

SPLIT = (TOTAL_ROWS // 1024) * 1024
TAIL = TOTAL_ROWS - SPLIT

NUM_CORES = 2
NUM_SUBCORES = 16
NUM_WORKERS = NUM_CORES * NUM_SUBCORES
ROWS_PER_W = BATCH // NUM_WORKERS
FLAT_PER_W = ROWS_PER_W * N_FIELDS
LANES = 16

_mesh = plsc.VectorSubcoreMesh(core_axis_name="c", subcore_axis_name="s")

_RETILE_BLOCK = 131072
_RETILE_GRID = -(-SPLIT // _RETILE_BLOCK)


def _retile_body(src_ref, dst_ref):
    dst_ref[...] = src_ref[0]


_w_linearize = pl.pallas_call(
    _retile_body,
    grid=(_RETILE_GRID,),
    out_shape=jax.ShapeDtypeStruct((SPLIT,), jnp.float32),
    in_specs=[pl.BlockSpec((1, _RETILE_BLOCK), lambda i: (0, i))],
    out_specs=pl.BlockSpec((_RETILE_BLOCK,), lambda i: (i,)),
)


def _idx_body(dataT_ref, idx_ref, tdelta_ref):
    for f in range(N_FIELDS):
        row = dataT_ref[f]
        idx_ref[pl.ds(f * BATCH, BATCH)] = jnp.minimum(
            row + f * FIELD_DIM, SPLIT - 1
        )
    row_last = dataT_ref[N_FIELDS - 1]
    tdelta_ref[...] = jnp.where(
        row_last >= FIELD_DIM - TAIL, row_last - (FIELD_DIM - TAIL) + 1, 0
    )


_idx_prep = pl.pallas_call(
    _idx_body,
    out_shape=(
        jax.ShapeDtypeStruct((N_FIELDS * BATCH,), jnp.int32),
        jax.ShapeDtypeStruct((BATCH,), jnp.int32),
    ),
)


@functools.partial(
    pl.kernel,
    mesh=_mesh,
    out_type=jax.ShapeDtypeStruct((BATCH,), jnp.float32),
    compiler_params=pltpu.CompilerParams(needs_layout_passes=False),
    scratch_types=[
        pltpu.VMEM((FLAT_PER_W,), jnp.int32),
        pltpu.VMEM((FLAT_PER_W,), jnp.float32),
        pltpu.VMEM((ROWS_PER_W,), jnp.int32),
        pltpu.VMEM((TAIL,), jnp.float32),
        pltpu.VMEM((ROWS_PER_W,), jnp.float32),
        pltpu.VMEM((LANES,), jnp.float32),
        pltpu.SemaphoreType.DMA,
        pltpu.SemaphoreType.DMA,
    ],
)
def _lr_sc(idx_hbm, tdelta_hbm, w_hbm, wtail_hbm, bias_hbm, out_hbm,
           idx_v, vals_v, tdelta_v, wtail_v, out_v, bias_v, sem, gsem):
    wid = lax.axis_index("s") * NUM_CORES + lax.axis_index("c")
    rbase = wid * ROWS_PER_W

    stages = [
        pltpu.make_async_copy(
            idx_hbm.at[pl.ds(f * BATCH + rbase, ROWS_PER_W)],
            idx_v.at[pl.ds(f * ROWS_PER_W, ROWS_PER_W)],
            sem,
        )
        for f in range(N_FIELDS)
    ]
    for c in stages:
        c.start()
    pltpu.sync_copy(tdelta_hbm.at[pl.ds(rbase, ROWS_PER_W)], tdelta_v)
    pltpu.sync_copy(wtail_hbm, wtail_v)
    pltpu.sync_copy(bias_hbm, bias_v)
    for c in stages:
        c.wait()

    pltpu.async_copy(w_hbm.at[idx_v], vals_v, gsem).wait()

    lane = lax.iota(jnp.int32, LANES)
    bvec = bias_v[...]
    zero16 = jnp.zeros((LANES,), jnp.int32)

    def reduce_block(blk, carry):
        r0 = blk * LANES
        acc = bvec
        for f in range(N_FIELDS - 1):
            acc = acc + vals_v[pl.ds(f * ROWS_PER_W + r0, LANES)]
        g = vals_v[pl.ds((N_FIELDS - 1) * ROWS_PER_W + r0, LANES)]
        d = tdelta_v[pl.ds(r0, LANES)]
        tv = plsc.load_gather(wtail_v, [jnp.maximum(d - 1, zero16)])
        acc = acc + jnp.where(d > 0, tv, g)
        out_v[pl.ds(r0, LANES)] = 1.0 / (1.0 + jnp.exp(-acc))
        return carry

    lax.fori_loop(0, ROWS_PER_W // LANES, reduce_block, 0)

    pltpu.sync_copy(out_v, out_hbm.at[pl.ds(rbase, ROWS_PER_W)])


def kernel(data, W, bias):
    idx_safe, tdelta = _idx_prep(data.T)
    w_main = _w_linearize(W.T)
    w_tail = lax.slice(W, (SPLIT, 0), (TOTAL_ROWS, 1)).reshape(-1)
    bias16 = jnp.broadcast_to(bias.astype(jnp.float32), (LANES,))
    return _lr_sc(idx_safe, tdelta, w_main, w_tail, bias16)


# speedup vs baseline: 7.0377x; 1.0966x over previous
import functools

import jax
import jax.numpy as jnp
from jax import lax
from jax.experimental import pallas as pl
from jax.experimental.pallas import tpu as pltpu
from jax.experimental.pallas import tpu_sc as plsc

BATCH = 16384
N_FIELDS = 26
FIELD_DIM = 100000
TOTAL_ROWS = N_FIELDS * FIELD_DIM

SPLIT = (TOTAL_ROWS // 1024) * 1024
TAIL = TOTAL_ROWS - SPLIT

NUM_CORES = 2
NUM_SUBCORES = 16
NUM_WORKERS = NUM_CORES * NUM_SUBCORES
ROWS_PER_W = BATCH // NUM_WORKERS
FLAT_PER_W = ROWS_PER_W * N_FIELDS
LANES = 16

_mesh = plsc.VectorSubcoreMesh(core_axis_name="c", subcore_axis_name="s")

_RETILE_BLOCK = 262144
_RETILE_GRID = -(-SPLIT // _RETILE_BLOCK)


def _retile_body(src_ref, dst_ref):
    dst_ref[...] = src_ref[0]


_w_linearize = pl.pallas_call(
    _retile_body,
    grid=(_RETILE_GRID,),
    out_shape=jax.ShapeDtypeStruct((SPLIT,), jnp.float32),
    in_specs=[pl.BlockSpec((1, _RETILE_BLOCK), lambda i: (0, i))],
    out_specs=pl.BlockSpec((_RETILE_BLOCK,), lambda i: (i,)),
)


def _idx_body(dataT_ref, idx_ref, tdelta_ref):
    for f in range(N_FIELDS):
        row = dataT_ref[f]
        idx_ref[pl.ds(f * BATCH, BATCH)] = jnp.minimum(
            row + f * FIELD_DIM, SPLIT - 1
        )
    row_last = dataT_ref[N_FIELDS - 1]
    tdelta_ref[...] = jnp.where(
        row_last >= FIELD_DIM - TAIL, row_last - (FIELD_DIM - TAIL) + 1, 0
    )


_idx_prep = pl.pallas_call(
    _idx_body,
    out_shape=(
        jax.ShapeDtypeStruct((N_FIELDS * BATCH,), jnp.int32),
        jax.ShapeDtypeStruct((BATCH,), jnp.int32),
    ),
)


@functools.partial(
    pl.kernel,
    mesh=_mesh,
    out_type=jax.ShapeDtypeStruct((BATCH,), jnp.float32),
    compiler_params=pltpu.CompilerParams(needs_layout_passes=False),
    scratch_types=[
        pltpu.VMEM((FLAT_PER_W,), jnp.int32),
        pltpu.VMEM((FLAT_PER_W,), jnp.float32),
        pltpu.VMEM((ROWS_PER_W,), jnp.int32),
        pltpu.VMEM((TAIL,), jnp.float32),
        pltpu.VMEM((ROWS_PER_W,), jnp.float32),
        pltpu.VMEM((ROWS_PER_W,), jnp.float32),
        pltpu.VMEM((LANES,), jnp.float32),
        pltpu.SemaphoreType.DMA,
        pltpu.SemaphoreType.DMA,
        pltpu.SemaphoreType.DMA,
        pltpu.SemaphoreType.DMA,
    ],
)
def _lr_sc(idx_hbm, tdelta_hbm, w_hbm, wtail_hbm, bias_hbm, out_hbm,
           idx_v, vals_v, tdelta_v, wtail_v, acc_v, out_v, bias_v,
           sem_a, sem_b, gsem_a, gsem_b):
    wid = lax.axis_index("s") * NUM_CORES + lax.axis_index("c")
    rbase = wid * ROWS_PER_W
    half = N_FIELDS // 2
    hflat = half * ROWS_PER_W

    def stage(f, sem):
        return pltpu.make_async_copy(
            idx_hbm.at[pl.ds(f * BATCH + rbase, ROWS_PER_W)],
            idx_v.at[pl.ds(f * ROWS_PER_W, ROWS_PER_W)],
            sem,
        )

    wave_a = [stage(f, sem_a) for f in range(half)]
    wave_b = [stage(f, sem_b) for f in range(half, N_FIELDS)]
    for c in wave_a:
        c.start()
    for c in wave_b:
        c.start()
    for c in wave_a:
        c.wait()
    g_a = pltpu.async_copy(
        w_hbm.at[idx_v.at[pl.ds(0, hflat)]], vals_v.at[pl.ds(0, hflat)], gsem_a
    )
    pltpu.sync_copy(tdelta_hbm.at[pl.ds(rbase, ROWS_PER_W)], tdelta_v)
    pltpu.sync_copy(wtail_hbm, wtail_v)
    pltpu.sync_copy(bias_hbm, bias_v)
    for c in wave_b:
        c.wait()
    g_b = pltpu.async_copy(
        w_hbm.at[idx_v.at[pl.ds(hflat, hflat)]],
        vals_v.at[pl.ds(hflat, hflat)],
        gsem_b,
    )

    lane = lax.iota(jnp.int32, LANES)
    bvec = bias_v[...]
    zero16 = jnp.zeros((LANES,), jnp.int32)

    g_a.wait()

    def reduce_a(blk, carry):
        r0 = blk * LANES
        acc = bvec
        for f in range(half):
            acc = acc + vals_v[pl.ds(f * ROWS_PER_W + r0, LANES)]
        acc_v[pl.ds(r0, LANES)] = acc
        return carry

    lax.fori_loop(0, ROWS_PER_W // LANES, reduce_a, 0)

    g_b.wait()

    def reduce_b(blk, carry):
        r0 = blk * LANES
        acc = acc_v[pl.ds(r0, LANES)]
        for f in range(half, N_FIELDS - 1):
            acc = acc + vals_v[pl.ds(f * ROWS_PER_W + r0, LANES)]
        g = vals_v[pl.ds((N_FIELDS - 1) * ROWS_PER_W + r0, LANES)]
        d = tdelta_v[pl.ds(r0, LANES)]
        tv = plsc.load_gather(wtail_v, [jnp.maximum(d - 1, zero16)])
        acc = acc + jnp.where(d > 0, tv, g)
        out_v[pl.ds(r0, LANES)] = 1.0 / (1.0 + jnp.exp(-acc))
        return carry

    lax.fori_loop(0, ROWS_PER_W // LANES, reduce_b, 0)

    pltpu.sync_copy(out_v, out_hbm.at[pl.ds(rbase, ROWS_PER_W)])


def kernel(data, W, bias):
    idx_safe, tdelta = _idx_prep(data.T)
    w_main = _w_linearize(W.T)
    w_tail = lax.slice(W, (SPLIT, 0), (TOTAL_ROWS, 1)).reshape(-1)
    bias16 = jnp.broadcast_to(bias.astype(jnp.float32), (LANES,))
    return _lr_sc(idx_safe, tdelta, w_main, w_tail, bias16)


# speedup vs baseline: 7.1882x; 1.0214x over previous
import functools

import jax
import jax.numpy as jnp
from jax import lax
from jax.experimental import pallas as pl
from jax.experimental.pallas import tpu as pltpu
from jax.experimental.pallas import tpu_sc as plsc

BATCH = 16384
N_FIELDS = 26
FIELD_DIM = 100000
TOTAL_ROWS = N_FIELDS * FIELD_DIM

SPLIT = (TOTAL_ROWS // 1024) * 1024
TAIL = TOTAL_ROWS - SPLIT

NUM_CORES = 2
NUM_SUBCORES = 16
NUM_WORKERS = NUM_CORES * NUM_SUBCORES
ROWS_PER_W = BATCH // NUM_WORKERS
FLAT_PER_W = ROWS_PER_W * N_FIELDS
LANES = 16
BLKS = ROWS_PER_W // LANES
HALF = N_FIELDS // 2
HFLAT = HALF * ROWS_PER_W

_mesh = plsc.VectorSubcoreMesh(core_axis_name="c", subcore_axis_name="s")

_RETILE_BLOCK = 262144
_RETILE_GRID = -(-SPLIT // _RETILE_BLOCK)


def _retile_body(src_ref, dst_ref):
    dst_ref[...] = src_ref[0]


_w_linearize = pl.pallas_call(
    _retile_body,
    grid=(_RETILE_GRID,),
    out_shape=jax.ShapeDtypeStruct((SPLIT,), jnp.float32),
    in_specs=[pl.BlockSpec((1, _RETILE_BLOCK), lambda i: (0, i))],
    out_specs=pl.BlockSpec((_RETILE_BLOCK,), lambda i: (i,)),
)


@functools.partial(
    pl.kernel,
    mesh=_mesh,
    out_type=jax.ShapeDtypeStruct((BATCH,), jnp.float32),
    compiler_params=pltpu.CompilerParams(needs_layout_passes=False),
    scratch_types=[
        pltpu.VMEM((FLAT_PER_W,), jnp.int32),
        pltpu.VMEM((FLAT_PER_W,), jnp.float32),
        pltpu.VMEM((ROWS_PER_W,), jnp.int32),
        pltpu.VMEM((TAIL,), jnp.float32),
        pltpu.VMEM((ROWS_PER_W,), jnp.float32),
        pltpu.VMEM((ROWS_PER_W,), jnp.float32),
        pltpu.VMEM((LANES,), jnp.float32),
        pltpu.SemaphoreType.DMA,
        pltpu.SemaphoreType.DMA,
        pltpu.SemaphoreType.DMA,
        pltpu.SemaphoreType.DMA,
    ],
)
def _lr_sc(data_hbm, w_hbm, wtail_hbm, bias_hbm, out_hbm,
           idx_v, vals_v, tdelta_v, wtail_v, acc_v, out_v, bias_v,
           sem_a, sem_b, gsem_a, gsem_b):
    wid = lax.axis_index("s") * NUM_CORES + lax.axis_index("c")
    rbase = wid * ROWS_PER_W

    def stage(f, sem):
        return pltpu.make_async_copy(
            data_hbm.at[f, pl.ds(rbase, ROWS_PER_W)],
            idx_v.at[pl.ds(f * ROWS_PER_W, ROWS_PER_W)],
            sem,
        )

    wave_a = [stage(f, sem_a) for f in range(HALF)]
    wave_b = [stage(f, sem_b) for f in range(HALF, N_FIELDS)]
    for c in wave_a:
        c.start()
    for c in wave_b:
        c.start()

    split_hi = jnp.full((LANES,), SPLIT - 1, jnp.int32)

    def add_offsets(lo_f, hi_f):
        def body(i, carry):
            f = lo_f + i // BLKS
            p0 = f * ROWS_PER_W + (i % BLKS) * LANES
            v = idx_v[pl.ds(p0, LANES)]
            idx_v[pl.ds(p0, LANES)] = jnp.minimum(v + f * FIELD_DIM, split_hi)
            return carry
        lax.fori_loop(0, (hi_f - lo_f) * BLKS, body, 0)

    for c in wave_a:
        c.wait()
    add_offsets(0, HALF)
    g_a = pltpu.async_copy(
        w_hbm.at[idx_v.at[pl.ds(0, HFLAT)]], vals_v.at[pl.ds(0, HFLAT)], gsem_a
    )
    pltpu.sync_copy(wtail_hbm, wtail_v)
    pltpu.sync_copy(bias_hbm, bias_v)
    for c in wave_b:
        c.wait()
    add_offsets(HALF, N_FIELDS - 1)

    def last_field(blk, carry):
        p0 = (N_FIELDS - 1) * ROWS_PER_W + blk * LANES
        v = idx_v[pl.ds(p0, LANES)]
        tdelta_v[pl.ds(blk * LANES, LANES)] = jnp.where(
            v >= FIELD_DIM - TAIL, v - (FIELD_DIM - TAIL) + 1, 0
        )
        idx_v[pl.ds(p0, LANES)] = jnp.minimum(
            v + (N_FIELDS - 1) * FIELD_DIM, split_hi
        )
        return carry

    lax.fori_loop(0, BLKS, last_field, 0)
    g_b = pltpu.async_copy(
        w_hbm.at[idx_v.at[pl.ds(HFLAT, HFLAT)]],
        vals_v.at[pl.ds(HFLAT, HFLAT)],
        gsem_b,
    )

    bvec = bias_v[...]
    zero16 = jnp.zeros((LANES,), jnp.int32)

    g_a.wait()

    def reduce_a(blk, carry):
        r0 = blk * LANES
        acc = bvec
        for f in range(HALF):
            acc = acc + vals_v[pl.ds(f * ROWS_PER_W + r0, LANES)]
        acc_v[pl.ds(r0, LANES)] = acc
        return carry

    lax.fori_loop(0, BLKS, reduce_a, 0)

    g_b.wait()

    def reduce_b(blk, carry):
        r0 = blk * LANES
        acc = acc_v[pl.ds(r0, LANES)]
        for f in range(HALF, N_FIELDS - 1):
            acc = acc + vals_v[pl.ds(f * ROWS_PER_W + r0, LANES)]
        g = vals_v[pl.ds((N_FIELDS - 1) * ROWS_PER_W + r0, LANES)]
        d = tdelta_v[pl.ds(r0, LANES)]
        tv = plsc.load_gather(wtail_v, [jnp.maximum(d - 1, zero16)])
        acc = acc + jnp.where(d > 0, tv, g)
        out_v[pl.ds(r0, LANES)] = 1.0 / (1.0 + jnp.exp(-acc))
        return carry

    lax.fori_loop(0, BLKS, reduce_b, 0)

    pltpu.sync_copy(out_v, out_hbm.at[pl.ds(rbase, ROWS_PER_W)])


def kernel(data, W, bias):
    w_main = _w_linearize(W.T)
    w_tail = lax.slice(W, (SPLIT, 0), (TOTAL_ROWS, 1)).reshape(-1)
    bias16 = jnp.broadcast_to(bias.astype(jnp.float32), (LANES,))
    return _lr_sc(data.T, w_main, w_tail, bias16)


# speedup vs baseline: 7.7031x; 1.0716x over previous
import functools

import jax
import jax.numpy as jnp
from jax import lax
from jax.experimental import pallas as pl
from jax.experimental.pallas import tpu as pltpu
from jax.experimental.pallas import tpu_sc as plsc

BATCH = 16384
N_FIELDS = 26
FIELD_DIM = 100000
TOTAL_ROWS = N_FIELDS * FIELD_DIM

SPLIT = (TOTAL_ROWS // 1024) * 1024
TAIL = TOTAL_ROWS - SPLIT

NUM_CORES = 2
NUM_SUBCORES = 16
NUM_WORKERS = NUM_CORES * NUM_SUBCORES
ROWS_PER_W = BATCH // NUM_WORKERS
FLAT_PER_W = ROWS_PER_W * N_FIELDS
LANES = 16
BLKS = ROWS_PER_W // LANES
HALF = N_FIELDS // 2
HFLAT = HALF * ROWS_PER_W

_mesh = plsc.VectorSubcoreMesh(core_axis_name="c", subcore_axis_name="s")

_RETILE_BLOCK = 524288
_RETILE_GRID = -(-SPLIT // _RETILE_BLOCK)


def _retile_body(src_ref, dst_ref):
    dst_ref[...] = src_ref[0]


_w_linearize = pl.pallas_call(
    _retile_body,
    grid=(_RETILE_GRID,),
    out_shape=jax.ShapeDtypeStruct((SPLIT,), jnp.float32),
    in_specs=[pl.BlockSpec((1, _RETILE_BLOCK), lambda i: (0, i))],
    out_specs=pl.BlockSpec((_RETILE_BLOCK,), lambda i: (i,)),
)


@functools.partial(
    pl.kernel,
    mesh=_mesh,
    out_type=jax.ShapeDtypeStruct((BATCH,), jnp.float32),
    compiler_params=pltpu.CompilerParams(needs_layout_passes=False),
    scratch_types=[
        pltpu.VMEM((FLAT_PER_W,), jnp.int32),
        pltpu.VMEM((FLAT_PER_W,), jnp.float32),
        pltpu.VMEM((ROWS_PER_W,), jnp.int32),
        pltpu.VMEM((TAIL,), jnp.float32),
        pltpu.VMEM((ROWS_PER_W,), jnp.float32),
        pltpu.VMEM((ROWS_PER_W,), jnp.float32),
        pltpu.VMEM((LANES,), jnp.float32),
        pltpu.SemaphoreType.DMA,
        pltpu.SemaphoreType.DMA,
        pltpu.SemaphoreType.DMA,
        pltpu.SemaphoreType.DMA,
        pltpu.SemaphoreType.DMA,
        pltpu.SemaphoreType.DMA,
        pltpu.SemaphoreType.DMA,
        pltpu.SemaphoreType.DMA,
    ],
)
def _lr_sc(data_hbm, w_hbm, wtail_hbm, bias_hbm, out_hbm,
           idx_v, vals_v, tdelta_v, wtail_v, acc_v, out_v, bias_v,
           *sems):
    wid = lax.axis_index("s") * NUM_CORES + lax.axis_index("c")
    rbase = wid * ROWS_PER_W
    waves = [(0, 7), (7, 13), (13, 20), (20, 26)]
    ssems, gsems = sems[:4], sems[4:]

    def stage(f, sem):
        return pltpu.make_async_copy(
            data_hbm.at[f, pl.ds(rbase, ROWS_PER_W)],
            idx_v.at[pl.ds(f * ROWS_PER_W, ROWS_PER_W)],
            sem,
        )

    stage_waves = [
        [stage(f, ssems[k]) for f in range(lo, hi)]
        for k, (lo, hi) in enumerate(waves)
    ]
    for wave in stage_waves:
        for c in wave:
            c.start()

    split_hi = jnp.full((LANES,), SPLIT - 1, jnp.int32)

    def add_offsets(lo_f, hi_f):
        def body(i, carry):
            f = lo_f + i // BLKS
            p0 = f * ROWS_PER_W + (i % BLKS) * LANES
            v = idx_v[pl.ds(p0, LANES)]
            idx_v[pl.ds(p0, LANES)] = jnp.minimum(v + f * FIELD_DIM, split_hi)
            return carry
        lax.fori_loop(0, (hi_f - lo_f) * BLKS, body, 0)

    def last_field(blk, carry):
        p0 = (N_FIELDS - 1) * ROWS_PER_W + blk * LANES
        v = idx_v[pl.ds(p0, LANES)]
        tdelta_v[pl.ds(blk * LANES, LANES)] = jnp.where(
            v >= FIELD_DIM - TAIL, v - (FIELD_DIM - TAIL) + 1, 0
        )
        idx_v[pl.ds(p0, LANES)] = jnp.minimum(
            v + (N_FIELDS - 1) * FIELD_DIM, split_hi
        )
        return carry

    gathers = []
    for k, (lo, hi) in enumerate(waves):
        for c in stage_waves[k]:
            c.wait()
        if hi == N_FIELDS:
            add_offsets(lo, hi - 1)
            lax.fori_loop(0, BLKS, last_field, 0)
        else:
            add_offsets(lo, hi)
        flat0, flatn = lo * ROWS_PER_W, (hi - lo) * ROWS_PER_W
        gathers.append(pltpu.async_copy(
            w_hbm.at[idx_v.at[pl.ds(flat0, flatn)]],
            vals_v.at[pl.ds(flat0, flatn)],
            gsems[k],
        ))
        if k == 0:
            pltpu.sync_copy(wtail_hbm, wtail_v)
            pltpu.sync_copy(bias_hbm, bias_v)

    bvec = bias_v[...]
    zero16 = jnp.zeros((LANES,), jnp.int32)

    for k, (lo, hi) in enumerate(waves):
        gathers[k].wait()

        def reduce_wave(blk, carry, lo=lo, hi=hi, first=(k == 0),
                        last=(k == len(waves) - 1)):
            r0 = blk * LANES
            acc = bvec if first else acc_v[pl.ds(r0, LANES)]
            for f in range(lo, hi - 1 if last else hi):
                acc = acc + vals_v[pl.ds(f * ROWS_PER_W + r0, LANES)]
            if last:
                g = vals_v[pl.ds((N_FIELDS - 1) * ROWS_PER_W + r0, LANES)]
                d = tdelta_v[pl.ds(r0, LANES)]
                tv = plsc.load_gather(wtail_v, [jnp.maximum(d - 1, zero16)])
                acc = acc + jnp.where(d > 0, tv, g)
                out_v[pl.ds(r0, LANES)] = 1.0 / (1.0 + jnp.exp(-acc))
            else:
                acc_v[pl.ds(r0, LANES)] = acc
            return carry

        lax.fori_loop(0, BLKS, reduce_wave, 0)

    pltpu.sync_copy(out_v, out_hbm.at[pl.ds(rbase, ROWS_PER_W)])


def kernel(data, W, bias):
    w_main = _w_linearize(W.T)
    w_tail = lax.slice(W, (SPLIT, 0), (TOTAL_ROWS, 1)).reshape(-1)
    bias16 = jnp.broadcast_to(bias.astype(jnp.float32), (LANES,))
    return _lr_sc(data.T, w_main, w_tail, bias16)
